# baseline (speedup 1.0000x reference)
import functools

import jax
import jax.numpy as jnp
from jax import lax
from jax.experimental import pallas as pl
from jax.experimental.pallas import tpu as pltpu
from jax.experimental.pallas import tpu_sc as plsc

B = 16384
K = 20
S = 5
D = 32
H = 64

NC = 2
NS = 16
NW = NC * NS
ROWS = 2 * B
BPW = B // NW
BLK = 128
NBLK = BPW // BLK
CH = 16
SUBS = BLK // CH
G = CH * K
VP = 4


def _sc_weighted_embed(emb4, idx_a_t, idx_h_t, w_a_t, w_h_t):
  mesh = plsc.VectorSubcoreMesh(core_axis_name="c", subcore_axis_name="s")
  NSEG = (G + 127) // 128

  @functools.partial(
      pl.kernel,
      mesh=mesh,
      out_type=jax.ShapeDtypeStruct((ROWS // VP, VP * D), jnp.float32),
      scratch_types=[
          pltpu.VMEM((K, BLK), jnp.int32),
          pltpu.VMEM((K, BLK), jnp.float32),
          pltpu.VMEM((G,), jnp.int32),
          pltpu.VMEM((G,), jnp.int32),
          pltpu.VMEM((G, VP * D), jnp.float32),
          pltpu.VMEM((G, VP * D), jnp.float32),
          pltpu.VMEM((2 * CH // VP, VP * D), jnp.float32),
          pltpu.SemaphoreType.DMA,
          pltpu.SemaphoreType.DMA,
      ],
  )
  def body(emb_hbm, ia_hbm, ih_hbm, wa_hbm, wh_hbm, out_hbm, idx_v, w_v,
           flat_a, flat_b, rows_a, rows_b, acc_v, sem_a, sem_b):
    wid = lax.axis_index("s") * NC + lax.axis_index("c")
    base = wid * BPW
    slots = ((flat_a, rows_a, sem_a), (flat_b, rows_b, sem_b))

    def build_flat(s, flat):
      for k in range(K):
        flat[pl.ds(k * CH, 16)] = idx_v[k, pl.ds(s * CH, 16)] >> 2

    def fire(flat, rows, sem):
      cps = []
      o = 0
      for seg in range(NSEG):
        n = min(128, G - seg * 128)
        cps.append(pltpu.async_copy(
            emb_hbm.at[flat.at[pl.ds(o, n)]], rows.at[pl.ds(o, n)], sem))
        o += n
      return cps

    for side, (i_hbm, v_hbm) in enumerate(((ia_hbm, wa_hbm), (ih_hbm, wh_hbm))):

      def block_body(c, carry):
        b0 = base + c * BLK
        pltpu.sync_copy(i_hbm.at[:, pl.ds(b0, BLK)], idx_v)
        pltpu.sync_copy(v_hbm.at[:, pl.ds(b0, BLK)], w_v)

        build_flat(0, slots[0][0])
        cps = fire(*slots[0])
        for s in range(SUBS):
          flat, rows, sem = slots[s % 2]
          if s + 1 < SUBS:
            nxt = slots[(s + 1) % 2]
            build_flat(s + 1, nxt[0])
            nxt_cps = fire(*nxt)
          else:
            nxt_cps = None
          for cp in cps:
            cp.wait()

          r0 = (s % 2) * (CH // VP)
          zero = jnp.zeros((16,), jnp.float32)
          for r in range(CH // VP):
            for h in range(VP * D // 16):
              acc_v[r0 + r, pl.ds(16 * h, 16)] = zero

          def k_step(k, carry2):
            g = k * CH
            kv = w_v[k, pl.ds(s * CH, 16)]
            jv = idx_v[k, pl.ds(s * CH, 16)]
            for e in range(CH):
              ws = kv[e]
              off = (jv[e] & 3) << 5
              v0 = rows[g + e, pl.ds(off, 16)] * ws
              v1 = rows[g + e, pl.ds(off + 16, 16)] * ws
              d0 = (e % VP) * D
              plsc.addupdate(acc_v.at[r0 + e // VP, pl.ds(d0, 16)], v0)
              plsc.addupdate(acc_v.at[r0 + e // VP, pl.ds(d0 + 16, 16)], v1)
            return carry2

          lax.fori_loop(0, K, k_step, 0)
          if s % 2 == 1:
            row0 = pl.multiple_of(
                (side * B + b0 + (s - 1) * CH) // VP, 8)
            pltpu.sync_copy(
                acc_v, out_hbm.at[pl.ds(row0, 2 * CH // VP)])
          cps = nxt_cps
        return carry

      lax.fori_loop(0, NBLK, block_body, 0)

  return body(emb4, idx_a_t, idx_h_t, w_a_t, w_h_t)


BS = 2048
GRID = B // BS


def _tc_head(a_s2, a_we, h_s2, h_we, e_all, W_big, b_stat2, W1a, W1b, b1_2,
             W2, b2_2, W3, b3_2):
  def body(a_s, a_w, h_s, h_w, ea, eh, wb, bst, w1a, w1b, bb1, w2, bb2, w3,
           bb3, out):
    f32 = jnp.float32
    dot = lambda x, y: lax.dot_general(x, y, (((1,), (0,)), ((), ())),
                                       preferred_element_type=f32)
    pa = dot(a_s[...] * a_w[...], wb[...]) + ea[...]
    pa = pa + (jnp.sum(a_w[...], axis=1, keepdims=True) * (1.0 / S)) * bst[...]
    ph = dot(h_s[...] * h_w[...], wb[...]) + eh[...]
    ph = ph + (jnp.sum(h_w[...], axis=1, keepdims=True) * (1.0 / S)) * bst[...]
    h1 = jnp.maximum(dot(pa, w1a[...]) + dot(ph, w1b[...]) + bb1[...], 0.0)
    h2 = jnp.maximum(dot(h1, w2[...]) + bb2[...], 0.0)
    out[...] = dot(h2, w3[...]) + bb3[...]

  KS = K * S
  in_specs = [
      pl.BlockSpec((BS, KS), lambda i: (i, 0)),
      pl.BlockSpec((BS, KS), lambda i: (i, 0)),
      pl.BlockSpec((BS, KS), lambda i: (i, 0)),
      pl.BlockSpec((BS, KS), lambda i: (i, 0)),
      pl.BlockSpec((BS, D), lambda i: (i, 0)),
      pl.BlockSpec((BS, D), lambda i: (i + GRID, 0)),
      pl.BlockSpec((KS, D), lambda i: (0, 0)),
      pl.BlockSpec((1, D), lambda i: (0, 0)),
      pl.BlockSpec((D, H), lambda i: (0, 0)),
      pl.BlockSpec((D, H), lambda i: (0, 0)),
      pl.BlockSpec((1, H), lambda i: (0, 0)),
      pl.BlockSpec((H, H), lambda i: (0, 0)),
      pl.BlockSpec((1, H), lambda i: (0, 0)),
      pl.BlockSpec((H, 1), lambda i: (0, 0)),
      pl.BlockSpec((1, 1), lambda i: (0, 0)),
  ]
  return pl.pallas_call(
      body,
      grid=(GRID,),
      in_specs=in_specs,
      out_specs=pl.BlockSpec((BS, 1), lambda i: (i, 0)),
      out_shape=jax.ShapeDtypeStruct((B, 1), jnp.float32),
  )(a_s2, a_we, h_s2, h_we, e_all, e_all, W_big, b_stat2, W1a, W1b, b1_2,
    W2, b2_2, W3, b3_2)


def kernel(away_indices, home_indices, away_stats, home_stats, away_weights,
           home_weights, W_stat, b_stat, emb, W1, b1, W2, b2, W3, b3):
  e4 = _sc_weighted_embed(emb.reshape(-1, VP * D),
                          away_indices.T.astype(jnp.int32),
                          home_indices.T.astype(jnp.int32),
                          away_weights.T, home_weights.T)
  e_all = e4.reshape(ROWS, D)

  a_s2 = away_stats.reshape(B, K * S)
  h_s2 = home_stats.reshape(B, K * S)
  a_we = jnp.repeat(away_weights, S, axis=1)
  h_we = jnp.repeat(home_weights, S, axis=1)
  W_big = jnp.tile(W_stat, (K, 1))
  out = _tc_head(a_s2, a_we, h_s2, h_we, e_all, W_big, b_stat.reshape(1, D),
                 W1[:D], W1[D:], b1.reshape(1, H), W2, b2.reshape(1, H),
                 W3, b3.reshape(1, 1))
  return out[:, 0]
